# baseline (speedup 1.0000x reference)
import functools
import jax
import jax.numpy as jnp
from jax import lax
from jax.experimental import pallas as pl
from jax.experimental.pallas import tpu as pltpu
from jax.experimental.pallas import tpu_sc as plsc

N = 10000
E = 320000
D = 128
NC = 2
NS = 16
NW = NC * NS
CHUNK = 128
CPT = 80
NBUF = 2
NG = CPT // NBUF
E_PAD = NW * CPT * CHUNK
N_PAD = 10112
RPT = N_PAD // NS
X_PAD = 10240
OUT = 7
BN_EPS = 1e-5


@functools.cache
def _make_sc_segsum():
    mesh = plsc.VectorSubcoreMesh(core_axis_name="c", subcore_axis_name="s",
                                  num_cores=NC, num_subcores=NS)

    @functools.partial(
        pl.kernel,
        out_type=jax.ShapeDtypeStruct((NC, N_PAD, D), jnp.float32),
        mesh=mesh,
        scratch_types=[
            pltpu.VMEM((CPT, CHUNK), jnp.int32),
            pltpu.VMEM((NBUF, CHUNK), jnp.int32),
            pltpu.VMEM((NBUF, CHUNK), jnp.int32),
            pltpu.VMEM((NBUF, CHUNK, D), jnp.float32),
            pltpu.VMEM_SHARED((N_PAD, D), jnp.float32),
            pltpu.SemaphoreType.DMA((NBUF,)),
            pltpu.SemaphoreType.DMA((NBUF,)),
        ],
    )
    def sc_segsum(x_hbm, pck_hbm, zero_hbm, out_hbm,
                  pck_v, src_v, dst_v, rows_v, acc, gsem, ssem):
        c = lax.axis_index("c")
        s = lax.axis_index("s")
        wid = s * NC + c
        pltpu.sync_copy(zero_hbm.at[pl.ds(s * RPT, RPT)],
                        acc.at[pl.ds(s * RPT, RPT)])
        pltpu.sync_copy(pck_hbm.at[wid], pck_v)
        plsc.subcore_barrier()

        def unpack(j, b):
            for k in range(CHUNK // 16):
                v = pck_v[j, pl.ds(16 * k, 16)]
                src_v[b, pl.ds(16 * k, 16)] = lax.bitwise_and(v, 0xFFFF)
                dst_v[b, pl.ds(16 * k, 16)] = lax.shift_right_logical(v, 16)

        def gather(b):
            pltpu.async_copy(x_hbm.at[src_v.at[b]], rows_v.at[b], gsem.at[b])

        def gather_wait(b):
            pltpu.make_async_copy(x_hbm.at[src_v.at[b]], rows_v.at[b],
                                  gsem.at[b]).wait()

        def scatter(b):
            pltpu.async_copy(rows_v.at[b], acc.at[dst_v.at[b]], ssem.at[b],
                             add=True)

        def scatter_wait(b):
            pltpu.make_async_copy(rows_v.at[b], acc.at[dst_v.at[b]],
                                  ssem.at[b]).wait()

        for b in range(NBUF):
            unpack(b, b)
            gather(b)

        @pl.loop(0, CPT, step=NBUF)
        def _chunk(j0):
            for b in range(NBUF):
                gather_wait(b)
                scatter(b)
            for b in range(NBUF):
                jn = j0 + b + NBUF

                @pl.when(jn < CPT)
                def _():
                    scatter_wait(b)
                    unpack(jn, b)
                    gather(b)

        for b in range(NBUF):
            scatter_wait(b)

        plsc.subcore_barrier()
        pltpu.sync_copy(acc.at[pl.ds(s * RPT, RPT)],
                        out_hbm.at[c, pl.ds(s * RPT, RPT)])

    return sc_segsum


def _sc_segsum(x, pck, zeros_pad):
    return _make_sc_segsum()(x, pck, zeros_pad)


BL = 1000


BLP = 512


def _row_mask(i, val):
    rows = lax.broadcasted_iota(jnp.int32, val.shape, 0) + i * BLP
    return jnp.where(rows < N, val, 0.0)


def _pre_body(d_ref, w1p_ref, b1_ref, w2_ref, b2_ref, w3p_ref, b3_ref,
              w4_ref, b4_ref, o_ref):
    d = d_ref[...]
    a = jnp.maximum(jnp.dot(d, w1p_ref[...], preferred_element_type=jnp.float32)
                    + b1_ref[...], 0.0)
    x2 = jnp.maximum(jnp.dot(a, w2_ref[...], preferred_element_type=jnp.float32)
                     + b2_ref[...], 0.0)
    i1 = jnp.maximum(
        jnp.dot(d.astype(jnp.bfloat16), w3p_ref[...].astype(jnp.bfloat16),
                preferred_element_type=jnp.float32)
        + b3_ref[...], 0.0)
    i2 = jnp.maximum(jnp.dot(i1, w4_ref[...], preferred_element_type=jnp.float32)
                     + b4_ref[...], 0.0)
    o_ref[...] = _row_mask(pl.program_id(0), jnp.concatenate((i2, x2), axis=1))


def _pre_mlp(data, w1p, b1, w2, b2, w3p, b3, w4, b4):
    grid = (X_PAD // BLP,)
    return pl.pallas_call(
        _pre_body,
        grid=grid,
        in_specs=[
            pl.BlockSpec((BLP, 1024), lambda i: (i, 0)),
            pl.BlockSpec((1024, 16), lambda i: (0, 0)),
            pl.BlockSpec((1, 16), lambda i: (0, 0)),
            pl.BlockSpec((16, 64), lambda i: (0, 0)),
            pl.BlockSpec((1, 64), lambda i: (0, 0)),
            pl.BlockSpec((1024, 256), lambda i: (0, 0)),
            pl.BlockSpec((1, 256), lambda i: (0, 0)),
            pl.BlockSpec((256, 64), lambda i: (0, 0)),
            pl.BlockSpec((1, 64), lambda i: (0, 0)),
        ],
        out_specs=pl.BlockSpec((BLP, D), lambda i: (i, 0)),
        out_shape=jax.ShapeDtypeStruct((X_PAD, D), jnp.float32),
    )(data, w1p, b1, w2, b2, w3p, b3, w4, b4)


EROWS = E // CHUNK
EROWS_PAD = E_PAD // CHUNK
EBL = EROWS_PAD // 8


def _pck_body(e_ref, o_ref):
    i = pl.program_id(0)
    src = e_ref[0]
    dst = e_ref[1]
    pos = ((lax.broadcasted_iota(jnp.int32, src.shape, 0) + i * EBL) * CHUNK
           + lax.broadcasted_iota(jnp.int32, src.shape, 1))
    pad_src = N + pos % (X_PAD - N)
    pad_dst = pos % N
    src = jnp.where(pos < E, src, pad_src)
    dst = jnp.where(pos < E, dst, pad_dst)
    o_ref[...] = jnp.bitwise_or(src, jnp.left_shift(dst, 16))


def _pck_pack(edge_index):
    return pl.pallas_call(
        _pck_body,
        grid=(8,),
        in_specs=[pl.BlockSpec((2, EBL, CHUNK), lambda i: (0, i, 0))],
        out_specs=pl.BlockSpec((EBL, CHUNK), lambda i: (i, 0)),
        out_shape=jax.ShapeDtypeStruct((EROWS_PAD, CHUNK), jnp.int32),
    )(edge_index.reshape(2, EROWS, CHUNK))


def _bn(t, st_ref, gamma_ref, beta_ref):
    mean = st_ref[0:1, :] * (1.0 / N)
    var = st_ref[1:2, :] * (1.0 / N) - mean * mean
    inv = lax.rsqrt(var + BN_EPS)
    return (t - mean) * inv * gamma_ref[...] + beta_ref[...]


def _gin_phase0(j, x_ref, p0_ref, p1_ref, w1_ref, b1_ref, t_buf, st_ref):
    h = x_ref[...] + p0_ref[0] + p1_ref[0]
    t = jnp.dot(h, w1_ref[...], preferred_element_type=jnp.float32) + b1_ref[...]
    t = _row_mask(j, t)
    t_buf[pl.ds(j * BLP, BLP), :] = t

    @pl.when(j == 0)
    def _():
        st_ref[...] = jnp.zeros_like(st_ref)

    s1 = jnp.sum(t, axis=0, keepdims=True)
    s2 = jnp.sum(t * t, axis=0, keepdims=True)
    st_ref[...] += jnp.concatenate((s1, s2, jnp.zeros((6, D), jnp.float32)),
                                   axis=0)


def _gin_fused_body(x_ref, p0_ref, p1_ref, w1_ref, b1_ref, g_ref, be_ref,
                    w2_ref, b2_ref, r_ref, o_ref, t_buf, st_ref):
    ph = pl.program_id(0)
    j = pl.program_id(1)

    @pl.when(ph == 0)
    def _():
        _gin_phase0(j, x_ref, p0_ref, p1_ref, w1_ref, b1_ref, t_buf, st_ref)

    @pl.when(ph == 1)
    def _():
        t = t_buf[pl.ds(j * BLP, BLP), :]
        tn = jnp.maximum(_bn(t, st_ref, g_ref, be_ref), 0.0)
        o = (jnp.dot(tn, w2_ref[...], preferred_element_type=jnp.float32)
             + b2_ref[...] + r_ref[...])
        o_ref[...] = _row_mask(j, o)


def _gin_fused(x, parts, w1, b1, gamma, beta, w2, b2, res):
    grid = (2, X_PAD // BLP)
    return pl.pallas_call(
        _gin_fused_body,
        grid=grid,
        in_specs=[
            pl.BlockSpec((BLP, D), lambda i, j: (j * (1 - i), 0)),
            pl.BlockSpec((1, BLP, D), lambda i, j: (0, j * (1 - i), 0)),
            pl.BlockSpec((1, BLP, D), lambda i, j: (1, j * (1 - i), 0)),
            pl.BlockSpec((D, D), lambda i, j: (0, 0)),
            pl.BlockSpec((1, D), lambda i, j: (0, 0)),
            pl.BlockSpec((1, D), lambda i, j: (0, 0)),
            pl.BlockSpec((1, D), lambda i, j: (0, 0)),
            pl.BlockSpec((D, D), lambda i, j: (0, 0)),
            pl.BlockSpec((1, D), lambda i, j: (0, 0)),
            pl.BlockSpec((BLP, D), lambda i, j: (j * i, 0)),
        ],
        out_specs=pl.BlockSpec((BLP, D), lambda i, j: (j * i, 0)),
        out_shape=jax.ShapeDtypeStruct((X_PAD, D), jnp.float32),
        scratch_shapes=[
            pltpu.VMEM((X_PAD, D), jnp.float32),
            pltpu.VMEM((8, D), jnp.float32),
        ],
    )(x, parts, parts, w1, b1, gamma, beta, w2, b2, res)


def _final_fused_body(x_ref, p0_ref, p1_ref, w1_ref, b1_ref, g_ref, be_ref,
                      w2_ref, b2_ref, r0_ref, r1_ref, wp1_ref, bp1_ref,
                      wp2_ref, bp2_ref, o_ref, t_buf, st_ref):
    ph = pl.program_id(0)
    j = pl.program_id(1)

    @pl.when(ph == 0)
    def _():
        _gin_phase0(j, x_ref, p0_ref, p1_ref, w1_ref, b1_ref, t_buf, st_ref)

    @pl.when(ph == 1)
    def _():
        t = t_buf[pl.ds(j * BLP, BLP), :]
        tn = jnp.maximum(_bn(t, st_ref, g_ref, be_ref), 0.0)
        g1 = (jnp.dot(tn, w2_ref[...], preferred_element_type=jnp.float32)
              + b2_ref[...] + r0_ref[...] + r1_ref[...])
        a = jnp.maximum(jnp.dot(g1, wp1_ref[...],
                                preferred_element_type=jnp.float32)
                        + bp1_ref[...], 0.0)
        o = (jnp.dot(a, wp2_ref[...], preferred_element_type=jnp.float32)
             + bp2_ref[...])
        m = jnp.max(o, axis=1, keepdims=True)
        z = o - m
        lse = jnp.log(jnp.sum(jnp.exp(z), axis=1, keepdims=True))
        o_ref[...] = (z - lse)[:, :OUT]


def _final_fused(x, parts, w1, b1, gamma, beta, w2, b2, res0, res1,
                 wp1, bp1, wp2p, bp2p):
    grid = (2, X_PAD // BLP)
    return pl.pallas_call(
        _final_fused_body,
        grid=grid,
        in_specs=[
            pl.BlockSpec((BLP, D), lambda i, j: (j * (1 - i), 0)),
            pl.BlockSpec((1, BLP, D), lambda i, j: (0, j * (1 - i), 0)),
            pl.BlockSpec((1, BLP, D), lambda i, j: (1, j * (1 - i), 0)),
            pl.BlockSpec((D, D), lambda i, j: (0, 0)),
            pl.BlockSpec((1, D), lambda i, j: (0, 0)),
            pl.BlockSpec((1, D), lambda i, j: (0, 0)),
            pl.BlockSpec((1, D), lambda i, j: (0, 0)),
            pl.BlockSpec((D, D), lambda i, j: (0, 0)),
            pl.BlockSpec((1, D), lambda i, j: (0, 0)),
            pl.BlockSpec((BLP, D), lambda i, j: (j * i, 0)),
            pl.BlockSpec((BLP, D), lambda i, j: (j * i, 0)),
            pl.BlockSpec((D, 32), lambda i, j: (0, 0)),
            pl.BlockSpec((1, 32), lambda i, j: (0, 0)),
            pl.BlockSpec((32, D), lambda i, j: (0, 0)),
            pl.BlockSpec((1, D), lambda i, j: (0, 0)),
        ],
        out_specs=pl.BlockSpec((BLP, OUT), lambda i, j: (j * i, 0)),
        out_shape=jax.ShapeDtypeStruct((N, OUT), jnp.float32),
        scratch_shapes=[
            pltpu.VMEM((X_PAD, D), jnp.float32),
            pltpu.VMEM((8, D), jnp.float32),
        ],
    )(x, parts, parts, w1, b1, gamma, beta, w2, b2, res0, res1,
      wp1, bp1, wp2p, bp2p)


def kernel(data, edge_index,
           w_pre1, b_pre1, w_pre2, b_pre2, w_pre3, b_pre3, w_pre4, b_pre4,
           w_post1, b_post1, w_post2, b_post2,
           gin0_w1, gin0_b1, gin0_gamma, gin0_beta, gin0_w2, gin0_b2,
           gin1_w1, gin1_b1, gin1_gamma, gin1_beta, gin1_w2, gin1_b2):
    f32 = jnp.float32
    w1p = jnp.zeros((1024, 16), f32).at[1022:, :].set(w_pre1)
    w3p = jnp.zeros((1024, 256), f32).at[:1022, :].set(w_pre3)
    wp2p = jnp.zeros((32, D), f32).at[:, :7].set(w_post2)
    bp2p = jnp.full((D,), -1e30, f32).at[:7].set(b_post2).reshape(1, D)

    row = lambda b: b.reshape(1, -1)

    new_x = _pre_mlp(data, w1p, row(b_pre1), w_pre2, row(b_pre2),
                     w3p, row(b_pre3), w_pre4, row(b_pre4))

    pck = _pck_pack(edge_index).reshape(NW, CPT, CHUNK)
    zeros_pad = jnp.zeros((N_PAD, D), f32)

    parts0 = _sc_segsum(new_x, pck, zeros_pad)
    g0 = _gin_fused(new_x, parts0, gin0_w1, row(gin0_b1), row(gin0_gamma),
                    row(gin0_beta), gin0_w2, row(gin0_b2), new_x)

    parts1 = _sc_segsum(g0, pck, zeros_pad)
    return _final_fused(g0, parts1, gin1_w1, row(gin1_b1), row(gin1_gamma),
                        row(gin1_beta), gin1_w2, row(gin1_b2), g0, new_x,
                        w_post1, row(b_post1), wp2p, bp2p)


# speedup vs baseline: 1.0691x; 1.0691x over previous
import functools
import jax
import jax.numpy as jnp
from jax import lax
from jax.experimental import pallas as pl
from jax.experimental.pallas import tpu as pltpu
from jax.experimental.pallas import tpu_sc as plsc

N = 10000
E = 320000
D = 128
NC = 2
NS = 16
NW = NC * NS
CHUNK = 128
CPT = 80
NBUF = 2
NG = CPT // NBUF
E_PAD = NW * CPT * CHUNK
N_PAD = 10112
RPT = N_PAD // NS
X_PAD = 10240
OUT = 7
BN_EPS = 1e-5


@functools.cache
def _make_sc_segsum():
    mesh = plsc.VectorSubcoreMesh(core_axis_name="c", subcore_axis_name="s",
                                  num_cores=NC, num_subcores=NS)

    @functools.partial(
        pl.kernel,
        out_type=jax.ShapeDtypeStruct((NC, N_PAD, D), jnp.float32),
        mesh=mesh,
        scratch_types=[
            pltpu.VMEM((CPT, CHUNK), jnp.int32),
            pltpu.VMEM((NBUF, CHUNK), jnp.int32),
            pltpu.VMEM((NBUF, CHUNK), jnp.int32),
            pltpu.VMEM((NBUF, CHUNK, D), jnp.float32),
            pltpu.VMEM_SHARED((N_PAD, D), jnp.float32),
            pltpu.SemaphoreType.DMA((NBUF,)),
            pltpu.SemaphoreType.DMA((NBUF,)),
        ],
    )
    def sc_segsum(x_hbm, pck_hbm, zero_hbm, out_hbm,
                  pck_v, src_v, dst_v, rows_v, acc, gsem, ssem):
        c = lax.axis_index("c")
        s = lax.axis_index("s")
        wid = s * NC + c
        pltpu.sync_copy(zero_hbm.at[pl.ds(s * RPT, RPT)],
                        acc.at[pl.ds(s * RPT, RPT)])
        pltpu.sync_copy(pck_hbm.at[wid], pck_v)
        plsc.subcore_barrier()

        def unpack(j, b):
            for k in range(CHUNK // 16):
                v = pck_v[j, pl.ds(16 * k, 16)]
                src_v[b, pl.ds(16 * k, 16)] = lax.bitwise_and(v, 0xFFFF)
                dst_v[b, pl.ds(16 * k, 16)] = lax.shift_right_logical(v, 16)

        def gather(b):
            pltpu.async_copy(x_hbm.at[src_v.at[b]], rows_v.at[b], gsem.at[b])

        def gather_wait(b):
            pltpu.make_async_copy(x_hbm.at[src_v.at[b]], rows_v.at[b],
                                  gsem.at[b]).wait()

        def scatter(b):
            pltpu.async_copy(rows_v.at[b], acc.at[dst_v.at[b]], ssem.at[b],
                             add=True)

        def scatter_wait(b):
            pltpu.make_async_copy(rows_v.at[b], acc.at[dst_v.at[b]],
                                  ssem.at[b]).wait()

        for b in range(NBUF):
            unpack(b, b)
            gather(b)

        @pl.loop(0, CPT, step=NBUF)
        def _chunk(j0):
            for b in range(NBUF):
                gather_wait(b)
                scatter(b)
            for b in range(NBUF):
                jn = j0 + b + NBUF

                @pl.when(jn < CPT)
                def _():
                    scatter_wait(b)
                    unpack(jn, b)
                    gather(b)

        for b in range(NBUF):
            scatter_wait(b)

        plsc.subcore_barrier()
        pltpu.sync_copy(acc.at[pl.ds(s * RPT, RPT)],
                        out_hbm.at[c, pl.ds(s * RPT, RPT)])

    return sc_segsum


def _sc_segsum(x, pck, zeros_pad):
    return _make_sc_segsum()(x, pck, zeros_pad)


BL = 1000


BLP = 1024


def _row_mask(i, val):
    rows = lax.broadcasted_iota(jnp.int32, val.shape, 0) + i * BLP
    return jnp.where(rows < N, val, 0.0)


def _pre_body(d_ref, w1p_ref, b1_ref, w2_ref, b2_ref, w3p_ref, b3_ref,
              w4_ref, b4_ref, o_ref):
    d = d_ref[...]
    a = jnp.maximum(jnp.dot(d, w1p_ref[...], preferred_element_type=jnp.float32)
                    + b1_ref[...], 0.0)
    x2 = jnp.maximum(jnp.dot(a, w2_ref[...], preferred_element_type=jnp.float32)
                     + b2_ref[...], 0.0)
    i1 = jnp.maximum(
        jnp.dot(d.astype(jnp.bfloat16), w3p_ref[...].astype(jnp.bfloat16),
                preferred_element_type=jnp.float32)
        + b3_ref[...], 0.0)
    i2 = jnp.maximum(jnp.dot(i1, w4_ref[...], preferred_element_type=jnp.float32)
                     + b4_ref[...], 0.0)
    o_ref[...] = _row_mask(pl.program_id(0), jnp.concatenate((i2, x2), axis=1))


def _pre_mlp(data, w1p, b1, w2, b2, w3p, b3, w4, b4):
    grid = (X_PAD // BLP,)
    return pl.pallas_call(
        _pre_body,
        grid=grid,
        in_specs=[
            pl.BlockSpec((BLP, 1024), lambda i: (i, 0)),
            pl.BlockSpec((1024, 16), lambda i: (0, 0)),
            pl.BlockSpec((1, 16), lambda i: (0, 0)),
            pl.BlockSpec((16, 64), lambda i: (0, 0)),
            pl.BlockSpec((1, 64), lambda i: (0, 0)),
            pl.BlockSpec((1024, 256), lambda i: (0, 0)),
            pl.BlockSpec((1, 256), lambda i: (0, 0)),
            pl.BlockSpec((256, 64), lambda i: (0, 0)),
            pl.BlockSpec((1, 64), lambda i: (0, 0)),
        ],
        out_specs=pl.BlockSpec((BLP, D), lambda i: (i, 0)),
        out_shape=jax.ShapeDtypeStruct((X_PAD, D), jnp.float32),
    )(data, w1p, b1, w2, b2, w3p, b3, w4, b4)


EROWS = E // CHUNK
EROWS_PAD = E_PAD // CHUNK
EBL = EROWS_PAD // 8


def _pck_body(e_ref, o_ref):
    i = pl.program_id(0)
    src = e_ref[0]
    dst = e_ref[1]
    pos = ((lax.broadcasted_iota(jnp.int32, src.shape, 0) + i * EBL) * CHUNK
           + lax.broadcasted_iota(jnp.int32, src.shape, 1))
    pad_src = N + pos % (X_PAD - N)
    pad_dst = pos % N
    src = jnp.where(pos < E, src, pad_src)
    dst = jnp.where(pos < E, dst, pad_dst)
    o_ref[...] = jnp.bitwise_or(src, jnp.left_shift(dst, 16))


def _pck_pack(edge_index):
    return pl.pallas_call(
        _pck_body,
        grid=(8,),
        in_specs=[pl.BlockSpec((2, EBL, CHUNK), lambda i: (0, i, 0))],
        out_specs=pl.BlockSpec((EBL, CHUNK), lambda i: (i, 0)),
        out_shape=jax.ShapeDtypeStruct((EROWS_PAD, CHUNK), jnp.int32),
    )(edge_index.reshape(2, EROWS, CHUNK))


def _bn(t, st_ref, gamma_ref, beta_ref):
    mean = st_ref[0:1, :] * (1.0 / N)
    var = st_ref[1:2, :] * (1.0 / N) - mean * mean
    inv = lax.rsqrt(var + BN_EPS)
    return (t - mean) * inv * gamma_ref[...] + beta_ref[...]


def _gin_phase0(j, x_ref, p0_ref, p1_ref, w1_ref, b1_ref, t_buf, st_ref):
    h = x_ref[...] + p0_ref[0] + p1_ref[0]
    t = jnp.dot(h, w1_ref[...], preferred_element_type=jnp.float32) + b1_ref[...]
    t = _row_mask(j, t)
    t_buf[pl.ds(j * BLP, BLP), :] = t

    @pl.when(j == 0)
    def _():
        st_ref[...] = jnp.zeros_like(st_ref)

    s1 = jnp.sum(t, axis=0, keepdims=True)
    s2 = jnp.sum(t * t, axis=0, keepdims=True)
    st_ref[...] += jnp.concatenate((s1, s2, jnp.zeros((6, D), jnp.float32)),
                                   axis=0)


def _gin_fused_body(x_ref, p0_ref, p1_ref, w1_ref, b1_ref, g_ref, be_ref,
                    w2_ref, b2_ref, r_ref, o_ref, t_buf, st_ref):
    ph = pl.program_id(0)
    j = pl.program_id(1)

    @pl.when(ph == 0)
    def _():
        _gin_phase0(j, x_ref, p0_ref, p1_ref, w1_ref, b1_ref, t_buf, st_ref)

    @pl.when(ph == 1)
    def _():
        t = t_buf[pl.ds(j * BLP, BLP), :]
        tn = jnp.maximum(_bn(t, st_ref, g_ref, be_ref), 0.0)
        o = (jnp.dot(tn, w2_ref[...], preferred_element_type=jnp.float32)
             + b2_ref[...] + r_ref[...])
        o_ref[...] = _row_mask(j, o)


def _gin_fused(x, parts, w1, b1, gamma, beta, w2, b2, res):
    grid = (2, X_PAD // BLP)
    return pl.pallas_call(
        _gin_fused_body,
        grid=grid,
        in_specs=[
            pl.BlockSpec((BLP, D), lambda i, j: (j * (1 - i), 0)),
            pl.BlockSpec((1, BLP, D), lambda i, j: (0, j * (1 - i), 0)),
            pl.BlockSpec((1, BLP, D), lambda i, j: (1, j * (1 - i), 0)),
            pl.BlockSpec((D, D), lambda i, j: (0, 0)),
            pl.BlockSpec((1, D), lambda i, j: (0, 0)),
            pl.BlockSpec((1, D), lambda i, j: (0, 0)),
            pl.BlockSpec((1, D), lambda i, j: (0, 0)),
            pl.BlockSpec((D, D), lambda i, j: (0, 0)),
            pl.BlockSpec((1, D), lambda i, j: (0, 0)),
            pl.BlockSpec((BLP, D), lambda i, j: (j * i, 0)),
        ],
        out_specs=pl.BlockSpec((BLP, D), lambda i, j: (j * i, 0)),
        out_shape=jax.ShapeDtypeStruct((X_PAD, D), jnp.float32),
        scratch_shapes=[
            pltpu.VMEM((X_PAD, D), jnp.float32),
            pltpu.VMEM((8, D), jnp.float32),
        ],
    )(x, parts, parts, w1, b1, gamma, beta, w2, b2, res)


def _final_fused_body(x_ref, p0_ref, p1_ref, w1_ref, b1_ref, g_ref, be_ref,
                      w2_ref, b2_ref, r0_ref, r1_ref, wp1_ref, bp1_ref,
                      wp2_ref, bp2_ref, o_ref, t_buf, st_ref):
    ph = pl.program_id(0)
    j = pl.program_id(1)

    @pl.when(ph == 0)
    def _():
        _gin_phase0(j, x_ref, p0_ref, p1_ref, w1_ref, b1_ref, t_buf, st_ref)

    @pl.when(ph == 1)
    def _():
        t = t_buf[pl.ds(j * BLP, BLP), :]
        tn = jnp.maximum(_bn(t, st_ref, g_ref, be_ref), 0.0)
        g1 = (jnp.dot(tn, w2_ref[...], preferred_element_type=jnp.float32)
              + b2_ref[...] + r0_ref[...] + r1_ref[...])
        a = jnp.maximum(jnp.dot(g1, wp1_ref[...],
                                preferred_element_type=jnp.float32)
                        + bp1_ref[...], 0.0)
        o = (jnp.dot(a, wp2_ref[...], preferred_element_type=jnp.float32)
             + bp2_ref[...])
        m = jnp.max(o, axis=1, keepdims=True)
        z = o - m
        lse = jnp.log(jnp.sum(jnp.exp(z), axis=1, keepdims=True))
        o_ref[...] = (z - lse)[:, :OUT]


def _final_fused(x, parts, w1, b1, gamma, beta, w2, b2, res0, res1,
                 wp1, bp1, wp2p, bp2p):
    grid = (2, X_PAD // BLP)
    return pl.pallas_call(
        _final_fused_body,
        grid=grid,
        in_specs=[
            pl.BlockSpec((BLP, D), lambda i, j: (j * (1 - i), 0)),
            pl.BlockSpec((1, BLP, D), lambda i, j: (0, j * (1 - i), 0)),
            pl.BlockSpec((1, BLP, D), lambda i, j: (1, j * (1 - i), 0)),
            pl.BlockSpec((D, D), lambda i, j: (0, 0)),
            pl.BlockSpec((1, D), lambda i, j: (0, 0)),
            pl.BlockSpec((1, D), lambda i, j: (0, 0)),
            pl.BlockSpec((1, D), lambda i, j: (0, 0)),
            pl.BlockSpec((D, D), lambda i, j: (0, 0)),
            pl.BlockSpec((1, D), lambda i, j: (0, 0)),
            pl.BlockSpec((BLP, D), lambda i, j: (j * i, 0)),
            pl.BlockSpec((BLP, D), lambda i, j: (j * i, 0)),
            pl.BlockSpec((D, 32), lambda i, j: (0, 0)),
            pl.BlockSpec((1, 32), lambda i, j: (0, 0)),
            pl.BlockSpec((32, D), lambda i, j: (0, 0)),
            pl.BlockSpec((1, D), lambda i, j: (0, 0)),
        ],
        out_specs=pl.BlockSpec((BLP, OUT), lambda i, j: (j * i, 0)),
        out_shape=jax.ShapeDtypeStruct((N, OUT), jnp.float32),
        scratch_shapes=[
            pltpu.VMEM((X_PAD, D), jnp.float32),
            pltpu.VMEM((8, D), jnp.float32),
        ],
    )(x, parts, parts, w1, b1, gamma, beta, w2, b2, res0, res1,
      wp1, bp1, wp2p, bp2p)


def kernel(data, edge_index,
           w_pre1, b_pre1, w_pre2, b_pre2, w_pre3, b_pre3, w_pre4, b_pre4,
           w_post1, b_post1, w_post2, b_post2,
           gin0_w1, gin0_b1, gin0_gamma, gin0_beta, gin0_w2, gin0_b2,
           gin1_w1, gin1_b1, gin1_gamma, gin1_beta, gin1_w2, gin1_b2):
    f32 = jnp.float32
    w1p = jnp.zeros((1024, 16), f32).at[1022:, :].set(w_pre1)
    w3p = jnp.zeros((1024, 256), f32).at[:1022, :].set(w_pre3)
    wp2p = jnp.zeros((32, D), f32).at[:, :7].set(w_post2)
    bp2p = jnp.full((D,), -1e30, f32).at[:7].set(b_post2).reshape(1, D)

    row = lambda b: b.reshape(1, -1)

    new_x = _pre_mlp(data, w1p, row(b_pre1), w_pre2, row(b_pre2),
                     w3p, row(b_pre3), w_pre4, row(b_pre4))

    pck = _pck_pack(edge_index).reshape(NW, CPT, CHUNK)
    zeros_pad = jnp.zeros((N_PAD, D), f32)

    parts0 = _sc_segsum(new_x, pck, zeros_pad)
    g0 = _gin_fused(new_x, parts0, gin0_w1, row(gin0_b1), row(gin0_gamma),
                    row(gin0_beta), gin0_w2, row(gin0_b2), new_x)

    parts1 = _sc_segsum(g0, pck, zeros_pad)
    return _final_fused(g0, parts1, gin1_w1, row(gin1_b1), row(gin1_gamma),
                        row(gin1_beta), gin1_w2, row(gin1_b2), g0, new_x,
                        w_post1, row(b_post1), wp2p, bp2p)


# speedup vs baseline: 1.0699x; 1.0008x over previous
import functools
import jax
import jax.numpy as jnp
from jax import lax
from jax.experimental import pallas as pl
from jax.experimental.pallas import tpu as pltpu
from jax.experimental.pallas import tpu_sc as plsc

N = 10000
E = 320000
D = 128
NC = 2
NS = 16
NW = NC * NS
CHUNK = 128
CPT = 80
NBUF = 2
E_PAD = NW * CPT * CHUNK
N_PAD = 10112
RPT = N_PAD // NS
X_PAD = 10240
OUT = 7
BN_EPS = 1e-5


@functools.cache
def _make_sc_segsum():
    mesh = plsc.VectorSubcoreMesh(core_axis_name="c", subcore_axis_name="s",
                                  num_cores=NC, num_subcores=NS)

    @functools.partial(
        pl.kernel,
        out_type=jax.ShapeDtypeStruct((NC, N_PAD, D), jnp.float32),
        mesh=mesh,
        scratch_types=[
            pltpu.VMEM((CPT, CHUNK), jnp.int32),
            pltpu.VMEM((NBUF, CHUNK), jnp.int32),
            pltpu.VMEM((NBUF, CHUNK), jnp.int32),
            pltpu.VMEM((NBUF, CHUNK, D), jnp.float32),
            pltpu.VMEM_SHARED((N_PAD, D), jnp.float32),
            pltpu.SemaphoreType.DMA((NBUF,)),
            pltpu.SemaphoreType.DMA((NBUF,)),
        ],
    )
    def sc_segsum(x_hbm, pck_hbm, zero_hbm, out_hbm,
                  pck_v, src_v, dst_v, rows_v, acc, gsem, ssem):
        c = lax.axis_index("c")
        s = lax.axis_index("s")
        wid = s * NC + c
        pltpu.sync_copy(zero_hbm.at[pl.ds(s * RPT, RPT)],
                        acc.at[pl.ds(s * RPT, RPT)])
        pltpu.sync_copy(pck_hbm.at[wid], pck_v)
        plsc.subcore_barrier()

        def unpack(j, b):
            for k in range(CHUNK // 16):
                v = pck_v[j, pl.ds(16 * k, 16)]
                src_v[b, pl.ds(16 * k, 16)] = lax.bitwise_and(v, 0xFFFF)
                dst_v[b, pl.ds(16 * k, 16)] = lax.shift_right_logical(v, 16)

        def gather(b):
            pltpu.async_copy(x_hbm.at[src_v.at[b]], rows_v.at[b], gsem.at[b])

        def gather_wait(b):
            pltpu.make_async_copy(x_hbm.at[src_v.at[b]], rows_v.at[b],
                                  gsem.at[b]).wait()

        def scatter(b):
            pltpu.async_copy(rows_v.at[b], acc.at[dst_v.at[b]], ssem.at[b],
                             add=True)

        def scatter_wait(b):
            pltpu.make_async_copy(rows_v.at[b], acc.at[dst_v.at[b]],
                                  ssem.at[b]).wait()

        for b in range(NBUF):
            unpack(b, b)
            gather(b)

        @pl.loop(0, CPT, step=NBUF)
        def _chunk(j0):
            for b in range(NBUF):
                gather_wait(b)
                scatter(b)
            for b in range(NBUF):
                jn = j0 + b + NBUF

                @pl.when(jn < CPT)
                def _():
                    scatter_wait(b)
                    unpack(jn, b)
                    gather(b)

        for b in range(NBUF):
            scatter_wait(b)

        plsc.subcore_barrier()
        pltpu.sync_copy(acc.at[pl.ds(s * RPT, RPT)],
                        out_hbm.at[c, pl.ds(s * RPT, RPT)])

    return sc_segsum


def _sc_segsum(x, pck, zeros_pad):
    return _make_sc_segsum()(x, pck, zeros_pad)


BLP = 1024


def _row_mask(i, val):
    rows = lax.broadcasted_iota(jnp.int32, val.shape, 0) + i * BLP
    return jnp.where(rows < N, val, 0.0)


def _pre_body(d_ref, w1p_ref, b1_ref, w2_ref, b2_ref, w3p_ref, b3_ref,
              w4_ref, b4_ref, o_ref):
    d = d_ref[...]
    a = jnp.maximum(jnp.dot(d, w1p_ref[...], preferred_element_type=jnp.float32)
                    + b1_ref[...], 0.0)
    x2 = jnp.maximum(jnp.dot(a, w2_ref[...], preferred_element_type=jnp.float32)
                     + b2_ref[...], 0.0)
    i1 = jnp.maximum(
        jnp.dot(d.astype(jnp.bfloat16), w3p_ref[...].astype(jnp.bfloat16),
                preferred_element_type=jnp.float32)
        + b3_ref[...], 0.0)
    i2 = jnp.maximum(jnp.dot(i1, w4_ref[...], preferred_element_type=jnp.float32)
                     + b4_ref[...], 0.0)
    o_ref[...] = _row_mask(pl.program_id(0), jnp.concatenate((i2, x2), axis=1))


def _pre_mlp(data, w1p, b1, w2, b2, w3p, b3, w4, b4):
    grid = (X_PAD // BLP,)
    return pl.pallas_call(
        _pre_body,
        grid=grid,
        in_specs=[
            pl.BlockSpec((BLP, 1024), lambda i: (i, 0)),
            pl.BlockSpec((1024, 16), lambda i: (0, 0)),
            pl.BlockSpec((1, 16), lambda i: (0, 0)),
            pl.BlockSpec((16, 64), lambda i: (0, 0)),
            pl.BlockSpec((1, 64), lambda i: (0, 0)),
            pl.BlockSpec((1024, 256), lambda i: (0, 0)),
            pl.BlockSpec((1, 256), lambda i: (0, 0)),
            pl.BlockSpec((256, 64), lambda i: (0, 0)),
            pl.BlockSpec((1, 64), lambda i: (0, 0)),
        ],
        out_specs=pl.BlockSpec((BLP, D), lambda i: (i, 0)),
        out_shape=jax.ShapeDtypeStruct((X_PAD, D), jnp.float32),
    )(data, w1p, b1, w2, b2, w3p, b3, w4, b4)


EROWS = E // CHUNK
EROWS_PAD = E_PAD // CHUNK
EBL = EROWS_PAD // 8


def _pck_body(e_ref, o_ref):
    i = pl.program_id(0)
    src = e_ref[0]
    dst = e_ref[1]
    pos = ((lax.broadcasted_iota(jnp.int32, src.shape, 0) + i * EBL) * CHUNK
           + lax.broadcasted_iota(jnp.int32, src.shape, 1))
    pad_src = N + pos % (X_PAD - N)
    pad_dst = pos % N
    src = jnp.where(pos < E, src, pad_src)
    dst = jnp.where(pos < E, dst, pad_dst)
    o_ref[...] = jnp.bitwise_or(src, jnp.left_shift(dst, 16))


def _pck_pack(edge_index):
    return pl.pallas_call(
        _pck_body,
        grid=(8,),
        in_specs=[pl.BlockSpec((2, EBL, CHUNK), lambda i: (0, i, 0))],
        out_specs=pl.BlockSpec((EBL, CHUNK), lambda i: (i, 0)),
        out_shape=jax.ShapeDtypeStruct((EROWS_PAD, CHUNK), jnp.int32),
    )(edge_index.reshape(2, EROWS, CHUNK))


def _bn(t, st_ref, gamma_ref, beta_ref):
    mean = st_ref[0:1, :] * (1.0 / N)
    var = st_ref[1:2, :] * (1.0 / N) - mean * mean
    inv = lax.rsqrt(var + BN_EPS)
    return (t - mean) * inv * gamma_ref[...] + beta_ref[...]


def _gin_phase0(j, x_ref, p0_ref, p1_ref, w1_ref, b1_ref, t_buf, st_ref):
    h = x_ref[...] + p0_ref[0] + p1_ref[0]
    t = jnp.dot(h, w1_ref[...], preferred_element_type=jnp.float32) + b1_ref[...]
    t = _row_mask(j, t)
    t_buf[pl.ds(j * BLP, BLP), :] = t

    @pl.when(j == 0)
    def _():
        st_ref[...] = jnp.zeros_like(st_ref)

    s1 = jnp.sum(t, axis=0, keepdims=True)
    s2 = jnp.sum(t * t, axis=0, keepdims=True)
    st_ref[...] += jnp.concatenate((s1, s2, jnp.zeros((6, D), jnp.float32)),
                                   axis=0)


def _gin_fused_body(x_ref, p0_ref, p1_ref, w1_ref, b1_ref, g_ref, be_ref,
                    w2_ref, b2_ref, r_ref, o_ref, t_buf, st_ref):
    ph = pl.program_id(0)
    j = pl.program_id(1)

    @pl.when(ph == 0)
    def _():
        _gin_phase0(j, x_ref, p0_ref, p1_ref, w1_ref, b1_ref, t_buf, st_ref)

    @pl.when(ph == 1)
    def _():
        t = t_buf[pl.ds(j * BLP, BLP), :]
        tn = jnp.maximum(_bn(t, st_ref, g_ref, be_ref), 0.0)
        o = (jnp.dot(tn, w2_ref[...], preferred_element_type=jnp.float32)
             + b2_ref[...] + r_ref[...])
        o_ref[...] = _row_mask(j, o)


def _gin_fused(x, parts, w1, b1, gamma, beta, w2, b2, res):
    grid = (2, X_PAD // BLP)
    return pl.pallas_call(
        _gin_fused_body,
        grid=grid,
        in_specs=[
            pl.BlockSpec((BLP, D), lambda i, j: (j * (1 - i), 0)),
            pl.BlockSpec((1, BLP, D), lambda i, j: (0, j * (1 - i), 0)),
            pl.BlockSpec((1, BLP, D), lambda i, j: (1, j * (1 - i), 0)),
            pl.BlockSpec((D, D), lambda i, j: (0, 0)),
            pl.BlockSpec((1, D), lambda i, j: (0, 0)),
            pl.BlockSpec((1, D), lambda i, j: (0, 0)),
            pl.BlockSpec((1, D), lambda i, j: (0, 0)),
            pl.BlockSpec((D, D), lambda i, j: (0, 0)),
            pl.BlockSpec((1, D), lambda i, j: (0, 0)),
            pl.BlockSpec((BLP, D), lambda i, j: (j * i, 0)),
        ],
        out_specs=pl.BlockSpec((BLP, D), lambda i, j: (j * i, 0)),
        out_shape=jax.ShapeDtypeStruct((X_PAD, D), jnp.float32),
        scratch_shapes=[
            pltpu.VMEM((X_PAD, D), jnp.float32),
            pltpu.VMEM((8, D), jnp.float32),
        ],
    )(x, parts, parts, w1, b1, gamma, beta, w2, b2, res)


def _final_fused_body(x_ref, p0_ref, p1_ref, w1_ref, b1_ref, g_ref, be_ref,
                      w2_ref, b2_ref, r0_ref, r1_ref, wp1_ref, bp1_ref,
                      wp2_ref, bp2_ref, o_ref, t_buf, st_ref):
    ph = pl.program_id(0)
    j = pl.program_id(1)

    @pl.when(ph == 0)
    def _():
        _gin_phase0(j, x_ref, p0_ref, p1_ref, w1_ref, b1_ref, t_buf, st_ref)

    @pl.when(ph == 1)
    def _():
        t = t_buf[pl.ds(j * BLP, BLP), :]
        tn = jnp.maximum(_bn(t, st_ref, g_ref, be_ref), 0.0)
        g1 = (jnp.dot(tn, w2_ref[...], preferred_element_type=jnp.float32)
              + b2_ref[...] + r0_ref[...] + r1_ref[...])
        a = jnp.maximum(jnp.dot(g1, wp1_ref[...],
                                preferred_element_type=jnp.float32)
                        + bp1_ref[...], 0.0)
        o = (jnp.dot(a, wp2_ref[...], preferred_element_type=jnp.float32)
             + bp2_ref[...])
        m = jnp.max(o, axis=1, keepdims=True)
        z = o - m
        lse = jnp.log(jnp.sum(jnp.exp(z), axis=1, keepdims=True))
        o_ref[...] = (z - lse)[:, :OUT]


def _final_fused(x, parts, w1, b1, gamma, beta, w2, b2, res0, res1,
                 wp1, bp1, wp2p, bp2p):
    grid = (2, X_PAD // BLP)
    return pl.pallas_call(
        _final_fused_body,
        grid=grid,
        in_specs=[
            pl.BlockSpec((BLP, D), lambda i, j: (j * (1 - i), 0)),
            pl.BlockSpec((1, BLP, D), lambda i, j: (0, j * (1 - i), 0)),
            pl.BlockSpec((1, BLP, D), lambda i, j: (1, j * (1 - i), 0)),
            pl.BlockSpec((D, D), lambda i, j: (0, 0)),
            pl.BlockSpec((1, D), lambda i, j: (0, 0)),
            pl.BlockSpec((1, D), lambda i, j: (0, 0)),
            pl.BlockSpec((1, D), lambda i, j: (0, 0)),
            pl.BlockSpec((D, D), lambda i, j: (0, 0)),
            pl.BlockSpec((1, D), lambda i, j: (0, 0)),
            pl.BlockSpec((BLP, D), lambda i, j: (j * i, 0)),
            pl.BlockSpec((BLP, D), lambda i, j: (j * i, 0)),
            pl.BlockSpec((D, 32), lambda i, j: (0, 0)),
            pl.BlockSpec((1, 32), lambda i, j: (0, 0)),
            pl.BlockSpec((32, D), lambda i, j: (0, 0)),
            pl.BlockSpec((1, D), lambda i, j: (0, 0)),
        ],
        out_specs=pl.BlockSpec((BLP, OUT), lambda i, j: (j * i, 0)),
        out_shape=jax.ShapeDtypeStruct((N, OUT), jnp.float32),
        scratch_shapes=[
            pltpu.VMEM((X_PAD, D), jnp.float32),
            pltpu.VMEM((8, D), jnp.float32),
        ],
    )(x, parts, parts, w1, b1, gamma, beta, w2, b2, res0, res1,
      wp1, bp1, wp2p, bp2p)


def kernel(data, edge_index,
           w_pre1, b_pre1, w_pre2, b_pre2, w_pre3, b_pre3, w_pre4, b_pre4,
           w_post1, b_post1, w_post2, b_post2,
           gin0_w1, gin0_b1, gin0_gamma, gin0_beta, gin0_w2, gin0_b2,
           gin1_w1, gin1_b1, gin1_gamma, gin1_beta, gin1_w2, gin1_b2):
    f32 = jnp.float32
    w1p = jnp.zeros((1024, 16), f32).at[1022:, :].set(w_pre1)
    w3p = jnp.zeros((1024, 256), f32).at[:1022, :].set(w_pre3)
    wp2p = jnp.zeros((32, D), f32).at[:, :7].set(w_post2)
    bp2p = jnp.full((D,), -1e30, f32).at[:7].set(b_post2).reshape(1, D)

    row = lambda b: b.reshape(1, -1)

    new_x = _pre_mlp(data, w1p, row(b_pre1), w_pre2, row(b_pre2),
                     w3p, row(b_pre3), w_pre4, row(b_pre4))

    pck = _pck_pack(edge_index).reshape(NW, CPT, CHUNK)
    zeros_pad = jnp.zeros((N_PAD, D), f32)

    parts0 = _sc_segsum(new_x, pck, zeros_pad)
    g0 = _gin_fused(new_x, parts0, gin0_w1, row(gin0_b1), row(gin0_gamma),
                    row(gin0_beta), gin0_w2, row(gin0_b2), new_x)

    parts1 = _sc_segsum(g0, pck, zeros_pad)
    return _final_fused(g0, parts1, gin1_w1, row(gin1_b1), row(gin1_gamma),
                        row(gin1_beta), gin1_w2, row(gin1_b2), g0, new_x,
                        w_post1, row(b_post1), wp2p, bp2p)
